# baseline (speedup 1.0000x reference)
import functools

import jax
import jax.numpy as jnp
from jax import lax
from jax.experimental import pallas as pl
from jax.experimental.pallas import tpu as pltpu

_EPS = 1e-5


def _affine_from(s_ref, q_ref, g_ref, b_ref, count, eps=_EPS):
    mean = s_ref[...] * (1.0 / count)
    var = jnp.maximum(q_ref[...] * (1.0 / count) - mean * mean, 0.0)
    inv = lax.rsqrt(var + eps)
    scale = g_ref[...] * inv
    shift = b_ref[...] - mean * scale
    return scale, shift


def _fused_kernel(x_ref, w1_ref, w2_ref, g1_ref, b1_ref, g2_ref, b2_ref,
                  out_ref,
                  y1_ref, y2_ref, xpad_ref, lhs_ref, stg_ref,
                  s1_ref, q1_ref, s2_ref, q2_ref, sem,
                  *, H, W, count, NB):
    ph = pl.program_id(0)
    b = pl.program_id(1)
    P, Bh, C = x_ref.shape
    Wp = W + 2

    @pl.when((ph == 0) & (b == 0))
    def _init():
        s1_ref[...] = jnp.zeros_like(s1_ref)
        q1_ref[...] = jnp.zeros_like(q1_ref)
        s2_ref[...] = jnp.zeros_like(s2_ref)
        q2_ref[...] = jnp.zeros_like(q2_ref)

    def conv(src_rows, w_ref, dst_ref, s_ref, q_ref):
        wb = w_ref[...].astype(jnp.bfloat16)
        xpad_ref[...] = jnp.zeros_like(xpad_ref)
        for r in range(H):
            base = Wp + 2 + Wp * r
            xpad_ref[base:base + W] = src_rows(r)
        sums = jnp.zeros((1, C), jnp.float32)
        sqs = jnp.zeros((1, C), jnp.float32)
        for r in range(H):
            for kh in range(3):
                for kw in range(3):
                    t_idx = kh * 3 + kw
                    a = Wp + 2 + Wp * r + Wp * (kh - 1) + (kw - 1)
                    lhs_ref[:, t_idx * C:(t_idx + 1) * C] = (
                        xpad_ref[a:a + W].reshape(W * Bh, C))
            acc = jnp.dot(lhs_ref[...], wb,
                          preferred_element_type=jnp.float32)
            sums = sums + jnp.sum(acc, axis=0, keepdims=True)
            sqs = sqs + jnp.sum(acc * acc, axis=0, keepdims=True)
            dst_ref[r * W:(r + 1) * W] = acc.reshape(W, Bh, C).astype(
                jnp.bfloat16)
        s_ref[...] += sums
        q_ref[...] += sqs

    @pl.when(ph == 0)
    def _phase0():
        conv(lambda r: x_ref[r * W:(r + 1) * W].astype(jnp.bfloat16),
             w1_ref, y1_ref.at[b], s1_ref, q1_ref)

    @pl.when(ph == 1)
    def _phase1():
        scale, shift = _affine_from(s1_ref, q1_ref, g1_ref, b1_ref, count)
        s3 = scale.reshape(1, 1, C)
        t3 = shift.reshape(1, 1, C)
        src = y1_ref.at[b]

        def rows(r):
            v = src[r * W:(r + 1) * W].astype(jnp.float32)
            return jnp.maximum(v * s3 + t3, 0.0).astype(jnp.bfloat16)

        conv(rows, w2_ref, y2_ref.at[b], s2_ref, q2_ref)

    @pl.when(ph == 2)
    def _phase2():
        scale, shift = _affine_from(s2_ref, q2_ref, g2_ref, b2_ref, count)
        s3 = scale.reshape(1, 1, C)
        t3 = shift.reshape(1, 1, C)

        def dma(chunk):
            return pltpu.make_async_copy(
                stg_ref, out_ref.at[:, pl.ds(chunk * Bh, Bh), :], sem)

        @pl.when(b > 0)
        def _wait_prev():
            dma(b - 1).wait()

        v = y2_ref[b].astype(jnp.float32)
        stg_ref[...] = jnp.maximum(v * s3 + t3, 0.0)
        dma(b).start()

        @pl.when(b == NB - 1)
        def _wait_last():
            dma(b).wait()


def _residual_block(xt, w1r, w2r, g1r, b1r, g2r, b2r, *, H, W):
    P, B, C = xt.shape
    NB = B // 16 if B % 16 == 0 else 1
    Bh = B // NB
    count = float(B * P)
    Wp = W + 2
    XP = Wp * (H + 2) + 8

    kern = functools.partial(_fused_kernel, H=H, W=W, count=count, NB=NB)
    grid_spec = pltpu.PrefetchScalarGridSpec(
        num_scalar_prefetch=0,
        grid=(3, NB),
        in_specs=[
            pl.BlockSpec((P, Bh, C),
                         lambda ph, b: (0, jnp.where(ph == 0, b, 0), 0)),
            pl.BlockSpec((9 * C, C), lambda ph, b: (0, 0)),
            pl.BlockSpec((9 * C, C), lambda ph, b: (0, 0)),
            pl.BlockSpec((1, C), lambda ph, b: (0, 0)),
            pl.BlockSpec((1, C), lambda ph, b: (0, 0)),
            pl.BlockSpec((1, C), lambda ph, b: (0, 0)),
            pl.BlockSpec((1, C), lambda ph, b: (0, 0)),
        ],
        out_specs=pl.BlockSpec(memory_space=pltpu.MemorySpace.HBM),
        scratch_shapes=[
            pltpu.VMEM((NB, P, Bh, C), jnp.bfloat16),
            pltpu.VMEM((NB, P, Bh, C), jnp.bfloat16),
            pltpu.VMEM((XP, Bh, C), jnp.bfloat16),
            pltpu.VMEM((W * Bh, 9 * C), jnp.bfloat16),
            pltpu.VMEM((P, Bh, C), jnp.float32),
            pltpu.VMEM((1, C), jnp.float32),
            pltpu.VMEM((1, C), jnp.float32),
            pltpu.VMEM((1, C), jnp.float32),
            pltpu.VMEM((1, C), jnp.float32),
            pltpu.SemaphoreType.DMA,
        ],
    )
    return pl.pallas_call(
        kern,
        out_shape=jax.ShapeDtypeStruct((P, B, C), jnp.float32),
        grid_spec=grid_spec,
        compiler_params=pltpu.CompilerParams(
            dimension_semantics=("arbitrary", "arbitrary"),
            vmem_limit_bytes=100 * 1024 * 1024,
        ),
    )(xt, w1r, w2r, g1r, b1r, g2r, b2r)


def kernel(x, w1, w2, g1, b1, g2, b2):
    B, C, H, W = x.shape
    P = H * W

    xt = jnp.transpose(x.reshape(B, C, P), (2, 0, 1))

    w1r = w1.reshape(9 * C, C).astype(jnp.float32)
    w2r = w2.reshape(9 * C, C).astype(jnp.float32)
    g1r = g1.astype(jnp.float32).reshape(1, C)
    b1r = b1.astype(jnp.float32).reshape(1, C)
    g2r = g2.astype(jnp.float32).reshape(1, C)
    b2r = b2.astype(jnp.float32).reshape(1, C)

    out = _residual_block(xt, w1r, w2r, g1r, b1r, g2r, b2r, H=H, W=W)

    return jnp.transpose(out, (1, 2, 0)).reshape(B, C, H, W).astype(x.dtype)


# speedup vs baseline: 1.1051x; 1.1051x over previous
import functools

import jax
import jax.numpy as jnp
from jax import lax
from jax.experimental import pallas as pl
from jax.experimental.pallas import tpu as pltpu

_EPS = 1e-5


def _affine_from(s_ref, q_ref, g_ref, b_ref, count, eps=_EPS):
    mean = s_ref[...] * (1.0 / count)
    var = jnp.maximum(q_ref[...] * (1.0 / count) - mean * mean, 0.0)
    inv = lax.rsqrt(var + eps)
    scale = g_ref[...] * inv
    shift = b_ref[...] - mean * scale
    return scale, shift


def _fused_kernel(x_ref, w1_ref, w2_ref, g1_ref, b1_ref, g2_ref, b2_ref,
                  out_ref,
                  y1_ref, y2_ref, xpad_ref, lhs_ref, stg_ref,
                  s1_ref, q1_ref, s2_ref, q2_ref, sem,
                  *, H, W, count, NB):
    ph = pl.program_id(0)
    b = pl.program_id(1)
    P, Bh, C = x_ref.shape
    Wp = W + 2

    @pl.when((ph == 0) & (b == 0))
    def _init():
        s1_ref[...] = jnp.zeros_like(s1_ref)
        q1_ref[...] = jnp.zeros_like(q1_ref)
        s2_ref[...] = jnp.zeros_like(s2_ref)
        q2_ref[...] = jnp.zeros_like(q2_ref)

    def conv(src_rows, w_ref, dst_ref, s_ref, q_ref):
        w = w_ref[...].astype(jnp.bfloat16)
        wrot = []
        for m in range(3):
            blocks = []
            for j in range(3):
                kh = (j - m + 1) % 3
                for kw in range(3):
                    t = kh * 3 + kw
                    blocks.append(w[t * C:(t + 1) * C])
            wrot.append(jnp.concatenate(blocks, axis=0))
        xpad_ref[...] = jnp.zeros_like(xpad_ref)
        for r in range(H):
            base = Wp + 2 + Wp * r
            xpad_ref[base:base + W] = src_rows(r)

        def put_slab(s):
            for kw in range(3):
                a = Wp + 2 + Wp * s + (kw - 1)
                lhs_ref[:, ((s % 3) * 3 + kw) * C:((s % 3) * 3 + kw + 1) * C
                        ] = xpad_ref[a:a + W].reshape(W * Bh, C)

        sums = jnp.zeros((1, C), jnp.float32)
        sqs = jnp.zeros((1, C), jnp.float32)
        for r in range(H):
            if r == 0:
                put_slab(-1)
                put_slab(0)
                put_slab(1)
            else:
                put_slab(r + 1)
            acc = jnp.dot(lhs_ref[...], wrot[r % 3],
                          preferred_element_type=jnp.float32)
            sums = sums + jnp.sum(acc, axis=0, keepdims=True)
            sqs = sqs + jnp.sum(acc * acc, axis=0, keepdims=True)
            dst_ref[r * W:(r + 1) * W] = acc.reshape(W, Bh, C).astype(
                jnp.bfloat16)
        s_ref[...] += sums
        q_ref[...] += sqs

    @pl.when(ph == 0)
    def _phase0():
        conv(lambda r: x_ref[r * W:(r + 1) * W].astype(jnp.bfloat16),
             w1_ref, y1_ref.at[b], s1_ref, q1_ref)

    @pl.when(ph == 1)
    def _phase1():
        scale, shift = _affine_from(s1_ref, q1_ref, g1_ref, b1_ref, count)
        s3 = scale.reshape(1, 1, C)
        t3 = shift.reshape(1, 1, C)
        src = y1_ref.at[b]

        def rows(r):
            v = src[r * W:(r + 1) * W].astype(jnp.float32)
            return jnp.maximum(v * s3 + t3, 0.0).astype(jnp.bfloat16)

        conv(rows, w2_ref, y2_ref.at[b], s2_ref, q2_ref)

    @pl.when(ph == 2)
    def _phase2():
        scale, shift = _affine_from(s2_ref, q2_ref, g2_ref, b2_ref, count)
        s3 = scale.reshape(1, 1, C)
        t3 = shift.reshape(1, 1, C)

        def dma(chunk):
            return pltpu.make_async_copy(
                stg_ref, out_ref.at[:, pl.ds(chunk * Bh, Bh), :], sem)

        @pl.when(b > 0)
        def _wait_prev():
            dma(b - 1).wait()

        v = y2_ref[b].astype(jnp.float32)
        stg_ref[...] = jnp.maximum(v * s3 + t3, 0.0)
        dma(b).start()

        @pl.when(b == NB - 1)
        def _wait_last():
            dma(b).wait()


def _residual_block(xt, w1r, w2r, g1r, b1r, g2r, b2r, *, H, W):
    P, B, C = xt.shape
    NB = B // 8 if B % 8 == 0 else 1
    Bh = B // NB
    count = float(B * P)
    Wp = W + 2
    XP = Wp * (H + 2) + 8

    kern = functools.partial(_fused_kernel, H=H, W=W, count=count, NB=NB)
    grid_spec = pltpu.PrefetchScalarGridSpec(
        num_scalar_prefetch=0,
        grid=(3, NB),
        in_specs=[
            pl.BlockSpec((P, Bh, C),
                         lambda ph, b: (0, jnp.where(ph == 0, b, 0), 0)),
            pl.BlockSpec((9 * C, C), lambda ph, b: (0, 0)),
            pl.BlockSpec((9 * C, C), lambda ph, b: (0, 0)),
            pl.BlockSpec((1, C), lambda ph, b: (0, 0)),
            pl.BlockSpec((1, C), lambda ph, b: (0, 0)),
            pl.BlockSpec((1, C), lambda ph, b: (0, 0)),
            pl.BlockSpec((1, C), lambda ph, b: (0, 0)),
        ],
        out_specs=pl.BlockSpec(memory_space=pltpu.MemorySpace.HBM),
        scratch_shapes=[
            pltpu.VMEM((NB, P, Bh, C), jnp.bfloat16),
            pltpu.VMEM((NB, P, Bh, C), jnp.bfloat16),
            pltpu.VMEM((XP, Bh, C), jnp.bfloat16),
            pltpu.VMEM((W * Bh, 9 * C), jnp.bfloat16),
            pltpu.VMEM((P, Bh, C), jnp.float32),
            pltpu.VMEM((1, C), jnp.float32),
            pltpu.VMEM((1, C), jnp.float32),
            pltpu.VMEM((1, C), jnp.float32),
            pltpu.VMEM((1, C), jnp.float32),
            pltpu.SemaphoreType.DMA,
        ],
    )
    return pl.pallas_call(
        kern,
        out_shape=jax.ShapeDtypeStruct((P, B, C), jnp.float32),
        grid_spec=grid_spec,
        compiler_params=pltpu.CompilerParams(
            dimension_semantics=("arbitrary", "arbitrary"),
            vmem_limit_bytes=100 * 1024 * 1024,
        ),
    )(xt, w1r, w2r, g1r, b1r, g2r, b2r)


def kernel(x, w1, w2, g1, b1, g2, b2):
    B, C, H, W = x.shape
    P = H * W

    xt = jnp.transpose(x.reshape(B, C, P), (2, 0, 1))

    w1r = w1.reshape(9 * C, C).astype(jnp.float32)
    w2r = w2.reshape(9 * C, C).astype(jnp.float32)
    g1r = g1.astype(jnp.float32).reshape(1, C)
    b1r = b1.astype(jnp.float32).reshape(1, C)
    g2r = g2.astype(jnp.float32).reshape(1, C)
    b2r = b2.astype(jnp.float32).reshape(1, C)

    out = _residual_block(xt, w1r, w2r, g1r, b1r, g2r, b2r, H=H, W=W)

    return jnp.transpose(out, (1, 2, 0)).reshape(B, C, H, W).astype(x.dtype)


# speedup vs baseline: 1.1553x; 1.0454x over previous
import functools

import jax
import jax.numpy as jnp
from jax import lax
from jax.experimental import pallas as pl
from jax.experimental.pallas import tpu as pltpu

_EPS = 1e-5


def _affine_from(s_ref, q_ref, g_ref, b_ref, count, eps=_EPS):
    mean = s_ref[...] * (1.0 / count)
    var = jnp.maximum(q_ref[...] * (1.0 / count) - mean * mean, 0.0)
    inv = lax.rsqrt(var + eps)
    scale = g_ref[...] * inv
    shift = b_ref[...] - mean * scale
    return scale, shift


def _fused_kernel(x_ref, w1_ref, w2_ref, g1_ref, b1_ref, g2_ref, b2_ref,
                  out_ref,
                  y1_ref, y2_ref, xpad_ref, lhs_ref, stg_ref,
                  s1_ref, q1_ref, s2_ref, q2_ref, sem,
                  *, H, W, count, NB):
    ph = pl.program_id(0)
    b = pl.program_id(1)
    P, Bh, C = x_ref.shape
    Wp = W + 2

    @pl.when((ph == 0) & (b == 0))
    def _init():
        s1_ref[...] = jnp.zeros_like(s1_ref)
        q1_ref[...] = jnp.zeros_like(q1_ref)
        s2_ref[...] = jnp.zeros_like(s2_ref)
        q2_ref[...] = jnp.zeros_like(q2_ref)

    def conv(src_rows, w_ref, dst_ref, s_ref, q_ref):
        w = w_ref[...].astype(jnp.bfloat16)
        wrot = []
        for m in range(3):
            blocks = []
            for j in range(3):
                kh = (j - m + 1) % 3
                for kw in range(3):
                    t = kh * 3 + kw
                    blocks.append(w[t * C:(t + 1) * C])
            wrot.append(jnp.concatenate(blocks, axis=0))
        xpad_ref[...] = jnp.zeros_like(xpad_ref)
        for r in range(H):
            base = Wp + 2 + Wp * r
            xpad_ref[base:base + W] = src_rows(r)

        def put_slab(s):
            for kw in range(3):
                a = Wp + 2 + Wp * s + (kw - 1)
                lhs_ref[:, ((s % 3) * 3 + kw) * C:((s % 3) * 3 + kw + 1) * C
                        ] = xpad_ref[a:a + W].reshape(W * Bh, C)

        sums = jnp.zeros((1, C), jnp.float32)
        sqs = jnp.zeros((1, C), jnp.float32)
        for r in range(H):
            if r == 0:
                put_slab(-1)
                put_slab(0)
                put_slab(1)
            else:
                put_slab(r + 1)
            acc = jnp.dot(lhs_ref[...], wrot[r % 3],
                          preferred_element_type=jnp.float32)
            sums = sums + jnp.sum(acc, axis=0, keepdims=True)
            sqs = sqs + jnp.sum(acc * acc, axis=0, keepdims=True)
            dst_ref[r * W:(r + 1) * W] = acc.reshape(W, Bh, C).astype(
                jnp.bfloat16)
        s_ref[...] += sums
        q_ref[...] += sqs

    @pl.when(ph == 0)
    def _phase0():
        conv(lambda r: x_ref[r * W:(r + 1) * W].astype(jnp.bfloat16),
             w1_ref, y1_ref.at[b], s1_ref, q1_ref)

    @pl.when(ph == 1)
    def _phase1():
        scale, shift = _affine_from(s1_ref, q1_ref, g1_ref, b1_ref, count)
        s3 = scale.reshape(1, 1, C)
        t3 = shift.reshape(1, 1, C)
        src = y1_ref.at[b]

        def rows(r):
            v = src[r * W:(r + 1) * W].astype(jnp.float32)
            return jnp.maximum(v * s3 + t3, 0.0).astype(jnp.bfloat16)

        conv(rows, w2_ref, y2_ref.at[b], s2_ref, q2_ref)

    @pl.when(ph == 2)
    def _phase2():
        scale, shift = _affine_from(s2_ref, q2_ref, g2_ref, b2_ref, count)
        s3 = scale.reshape(1, 1, C)
        t3 = shift.reshape(1, 1, C)

        def dma(chunk):
            return pltpu.make_async_copy(
                stg_ref, out_ref.at[:, pl.ds(chunk * Bh, Bh), :], sem)

        @pl.when(b > 0)
        def _wait_prev():
            dma(b - 1).wait()

        v = y2_ref[b].astype(jnp.float32)
        stg_ref[...] = jnp.maximum(v * s3 + t3, 0.0)
        dma(b).start()

        @pl.when(b == NB - 1)
        def _wait_last():
            dma(b).wait()


def _residual_block(xt, w1r, w2r, g1r, b1r, g2r, b2r, *, H, W):
    P, B, C = xt.shape
    NB = B // 16 if B % 16 == 0 else 1
    Bh = B // NB
    count = float(B * P)
    Wp = W + 2
    XP = Wp * (H + 2) + 8

    kern = functools.partial(_fused_kernel, H=H, W=W, count=count, NB=NB)
    grid_spec = pltpu.PrefetchScalarGridSpec(
        num_scalar_prefetch=0,
        grid=(3, NB),
        in_specs=[
            pl.BlockSpec((P, Bh, C),
                         lambda ph, b: (0, jnp.where(ph == 0, b, 0), 0)),
            pl.BlockSpec((9 * C, C), lambda ph, b: (0, 0)),
            pl.BlockSpec((9 * C, C), lambda ph, b: (0, 0)),
            pl.BlockSpec((1, C), lambda ph, b: (0, 0)),
            pl.BlockSpec((1, C), lambda ph, b: (0, 0)),
            pl.BlockSpec((1, C), lambda ph, b: (0, 0)),
            pl.BlockSpec((1, C), lambda ph, b: (0, 0)),
        ],
        out_specs=pl.BlockSpec(memory_space=pltpu.MemorySpace.HBM),
        scratch_shapes=[
            pltpu.VMEM((NB, P, Bh, C), jnp.bfloat16),
            pltpu.VMEM((NB, P, Bh, C), jnp.bfloat16),
            pltpu.VMEM((XP, Bh, C), jnp.bfloat16),
            pltpu.VMEM((W * Bh, 9 * C), jnp.bfloat16),
            pltpu.VMEM((P, Bh, C), jnp.float32),
            pltpu.VMEM((1, C), jnp.float32),
            pltpu.VMEM((1, C), jnp.float32),
            pltpu.VMEM((1, C), jnp.float32),
            pltpu.VMEM((1, C), jnp.float32),
            pltpu.SemaphoreType.DMA,
        ],
    )
    return pl.pallas_call(
        kern,
        out_shape=jax.ShapeDtypeStruct((P, B, C), jnp.float32),
        grid_spec=grid_spec,
        compiler_params=pltpu.CompilerParams(
            dimension_semantics=("arbitrary", "arbitrary"),
            vmem_limit_bytes=100 * 1024 * 1024,
        ),
    )(xt, w1r, w2r, g1r, b1r, g2r, b2r)


def kernel(x, w1, w2, g1, b1, g2, b2):
    B, C, H, W = x.shape
    P = H * W

    xt = jnp.transpose(x.reshape(B, C, P), (2, 0, 1))

    w1r = w1.reshape(9 * C, C).astype(jnp.float32)
    w2r = w2.reshape(9 * C, C).astype(jnp.float32)
    g1r = g1.astype(jnp.float32).reshape(1, C)
    b1r = b1.astype(jnp.float32).reshape(1, C)
    g2r = g2.astype(jnp.float32).reshape(1, C)
    b2r = b2.astype(jnp.float32).reshape(1, C)

    out = _residual_block(xt, w1r, w2r, g1r, b1r, g2r, b2r, H=H, W=W)

    return jnp.transpose(out, (1, 2, 0)).reshape(B, C, H, W).astype(x.dtype)
